# baseline (speedup 1.0000x reference)
import jax
import jax.numpy as jnp
from jax import lax
from jax.experimental import pallas as pl
from jax.experimental.pallas import tpu as pltpu
from jax.experimental.pallas import tpu_sc as plsc

D_MODEL = 64
SCALE = 8.0
NC, NS = 2, 16
NW = NC * NS
LANES = 16
GRP = 128
CHUNK = 512
GPC = CHUNK // GRP


def _emb_body(b_per_w, n_chunks):
    def body(x_hbm, table_hbm, out_hbm, idx_v, rows_v, sem_g):
        wid = lax.axis_index("s") * NC + lax.axis_index("c")
        base = wid * b_per_w
        pltpu.sync_copy(x_hbm.at[wid], idx_v)

        @pl.loop(0, n_chunks)
        def _chunk(g):
            descs = [
                pltpu.async_copy(
                    table_hbm.at[idx_v.at[g * GPC + k]],
                    rows_v.at[pl.ds(k * GRP, GRP)],
                    sem_g,
                )
                for k in range(GPC)
            ]
            for d in descs:
                d.wait()

            @pl.loop(0, CHUNK)
            def _scale(r):
                for j in range(D_MODEL // LANES):
                    sl = pl.ds(j * LANES, LANES)
                    rows_v[r, sl] = rows_v[r, sl] * SCALE

            pltpu.sync_copy(rows_v, out_hbm.at[pl.ds(base + g * CHUNK, CHUNK)])

    return body


def kernel(x, table):
    rows, cols = x.shape
    B = rows * cols
    b_per_w = B // NW
    n_chunks = b_per_w // CHUNK
    xr = x.reshape(NW, b_per_w // GRP, GRP)
    mesh = plsc.VectorSubcoreMesh(core_axis_name="c", subcore_axis_name="s")
    out = pl.kernel(
        _emb_body(b_per_w, n_chunks),
        out_type=jax.ShapeDtypeStruct((B, D_MODEL), jnp.float32),
        mesh=mesh,
        scratch_types=[
            pltpu.VMEM((b_per_w // GRP, GRP), jnp.int32),
            pltpu.VMEM((CHUNK, D_MODEL), jnp.float32),
            pltpu.SemaphoreType.DMA,
        ],
        compiler_params=pltpu.CompilerParams(use_tc_tiling_on_sc=False),
    )(xr, table)
    return out.reshape(rows, cols, D_MODEL)


# speedup vs baseline: 1.1202x; 1.1202x over previous
import jax
import jax.numpy as jnp
from jax import lax
from jax.experimental import pallas as pl
from jax.experimental.pallas import tpu as pltpu
from jax.experimental.pallas import tpu_sc as plsc

D_MODEL = 64
SCALE = 8.0
NC, NS = 2, 16
NW = NC * NS
LANES = 16
GRP = 128
NBUF = 8
LOOKAHEAD = 4


def _emb_body(b_per_w, n_groups):
    def body(x_hbm, table_hbm, out_hbm, idx_v, rows_v, sem_g, sem_o):
        wid = lax.axis_index("s") * NC + lax.axis_index("c")
        base = wid * b_per_w
        pltpu.sync_copy(x_hbm.at[wid], idx_v)

        def issue_gather(g, s):
            pltpu.async_copy(
                table_hbm.at[idx_v.at[g]], rows_v.at[s], sem_g.at[s]
            )

        def wait_gather(s):
            pltpu.make_async_copy(
                table_hbm.at[pl.ds(0, GRP)], rows_v.at[s], sem_g.at[s]
            ).wait()

        def issue_wb(g, s):
            pltpu.async_copy(
                rows_v.at[s],
                out_hbm.at[pl.ds(base + g * GRP, GRP)],
                sem_o.at[s],
            )

        def wait_wb(s):
            pltpu.make_async_copy(
                rows_v.at[s], out_hbm.at[pl.ds(0, GRP)], sem_o.at[s]
            ).wait()

        for s in range(LOOKAHEAD):
            issue_gather(s, s)

        @pl.loop(0, n_groups, step=NBUF)
        def _step(g0):
            for s in range(NBUF):
                g = g0 + s
                gi = g + LOOKAHEAD
                si = (s + LOOKAHEAD) % NBUF

                @pl.when(gi < n_groups)
                def _():
                    @pl.when(gi >= NBUF)
                    def _():
                        wait_wb(si)

                    issue_gather(gi, si)

                wait_gather(s)

                @pl.loop(0, GRP, unroll=4)
                def _scale(r):
                    for j in range(D_MODEL // LANES):
                        sl = pl.ds(j * LANES, LANES)
                        rows_v[s, r, sl] = rows_v[s, r, sl] * SCALE

                issue_wb(g, s)

        for s in range(NBUF):
            wait_wb(s)

    return body


def kernel(x, table):
    rows, cols = x.shape
    B = rows * cols
    b_per_w = B // NW
    n_groups = b_per_w // GRP
    xr = x.reshape(NW, n_groups, GRP)
    mesh = plsc.VectorSubcoreMesh(core_axis_name="c", subcore_axis_name="s")
    out = pl.kernel(
        _emb_body(b_per_w, n_groups),
        out_type=jax.ShapeDtypeStruct((B, D_MODEL), jnp.float32),
        mesh=mesh,
        scratch_types=[
            pltpu.VMEM((n_groups, GRP), jnp.int32),
            pltpu.VMEM((NBUF, GRP, D_MODEL), jnp.float32),
            pltpu.SemaphoreType.DMA((NBUF,)),
            pltpu.SemaphoreType.DMA((NBUF,)),
        ],
        compiler_params=pltpu.CompilerParams(use_tc_tiling_on_sc=False),
    )(xr, table)
    return out.reshape(rows, cols, D_MODEL)
